# Initial kernel scaffold; baseline (speedup 1.0000x reference)
import jax
import jax.numpy as jnp
from jax.experimental import pallas as pl


def kernel(x, edge_index, edge_attr, batch, Wq0, Wk0, Wv0, We0, Ws0, cWq, cWk, cWv, cWe, cWs, GP_W1, GP_b1, GP_W2, GP_b2, GB_W1, GB_b1, GB_W2, GB_b2, M_W0, M_b0, M_W1, M_b1, M_W2, M_b2, M_W3, M_b3):
    raise NotImplementedError("write your pallas kernel here")


# speedup vs baseline: 1.0008x; 1.0008x over previous
import jax
import jax.numpy as jnp
from jax.experimental import pallas as pl
from jax.experimental.pallas import tpu as pltpu

N = 10000
D = 64
G = 16


def _seg_softmax(s, ids, n):
    m = jax.ops.segment_max(s, ids, num_segments=n)
    m = jnp.where(jnp.isfinite(m), m, 0.0)
    ex = jnp.exp(s - m[ids])
    den = jax.ops.segment_sum(ex, ids, num_segments=n)
    return ex / (den[ids] + 1e-16)


def _conv(x, src, dst, ea, Wq, Wk, Wv, We, Ws):
    q = x @ Wq
    k = x @ Wk
    v = x @ Wv
    e = ea @ We
    kj = k[src] + e
    vj = v[src] + e
    score = jnp.sum(q[dst] * kj, axis=-1) / jnp.sqrt(float(D))
    alpha = _seg_softmax(score, dst, x.shape[0])
    agg = jax.ops.segment_sum(alpha[:, None] * vj, dst, num_segments=x.shape[0])
    return agg + x @ Ws


def _final_mlp_kernel(h_ref, w0_ref, b0_ref, w1_ref, b1_ref, w2_ref, b2_ref,
                      w3_ref, b3_ref, o_ref):
    h = h_ref[...]
    h = jnp.maximum(h @ w0_ref[...] + b0_ref[...], 0.0)
    h = jnp.maximum(h @ w1_ref[...] + b1_ref[...], 0.0)
    h = jnp.maximum(h @ w2_ref[...] + b2_ref[...], 0.0)
    o_ref[...] = h @ w3_ref[...] + b3_ref[...]


def kernel(x, edge_index, edge_attr, batch, Wq0, Wk0, Wv0, We0, Ws0, cWq, cWk,
           cWv, cWe, cWs, GP_W1, GP_b1, GP_W2, GP_b2, GB_W1, GB_b1, GB_W2,
           GB_b2, M_W0, M_b0, M_W1, M_b1, M_W2, M_b2, M_W3, M_b3):
    src = edge_index[0]
    dst = edge_index[1]
    out = jax.nn.elu(_conv(x, src, dst, edge_attr, Wq0, Wk0, Wv0, We0, Ws0))
    outs = [out]
    for i in range(5):
        out = jax.nn.elu(_conv(out, src, dst, edge_attr, cWq[i], cWk[i],
                               cWv[i], cWe[i], cWs[i]))
        outs.append(out)
    jk = jnp.max(jnp.stack(outs, axis=0), axis=0)

    def att_pool(W1, b1, W2, b2):
        gate = jax.nn.relu(jk @ W1 + b1) @ W2 + b2
        a = _seg_softmax(gate[:, 0], batch, G)
        return jax.ops.segment_sum(a[:, None] * jk, batch, num_segments=G)

    gP = att_pool(GP_W1, GP_b1, GP_W2, GP_b2)
    gB = att_pool(GB_W1, GB_b1, GB_W2, GB_b2)
    h = jnp.concatenate([gP, gB], axis=1)

    return pl.pallas_call(
        _final_mlp_kernel,
        out_shape=jax.ShapeDtypeStruct((G, 1), jnp.float32),
    )(h, M_W0, M_b0, M_W1, M_b1, M_W2, M_b2, M_W3, M_b3)


# speedup vs baseline: 10.1166x; 10.1080x over previous
import functools

import jax
import jax.numpy as jnp
from jax import lax
from jax.experimental import pallas as pl
from jax.experimental.pallas import tpu as pltpu
from jax.experimental.pallas import tpu_sc as plsc

_N = 10000
_E = 320000
_D = 64
_EDIM = 16
_G = 16
_NTILES = 32
_TPW = _E // _NTILES
_NB = 80
_NSTEP = _TPW // _NB
_NPAD = 10240
_RPT = _NPAD // 16
_ZR = 128


def _prep_body(x_ref, wq_ref, wk_ref, wv_ref, ws_ref, we_ref,
               kv_ref, qq_ref, s_ref):
    x = x_ref[...]
    q = x @ wq_ref[...]
    kv_ref[:, :_D] = x @ wk_ref[...]
    kv_ref[:, _D:] = x @ wv_ref[...]
    qq_ref[:, :_D] = q
    qq_ref[:, _D:_D + _EDIM] = lax.dot_general(
        q, we_ref[...], (((1,), (1,)), ((), ())))
    qq_ref[:, _D + _EDIM:] = jnp.zeros((x.shape[0], 128 - _D - _EDIM),
                                       jnp.float32)
    s_ref[...] = x @ ws_ref[...]


def _tc_prep(x, wq, wk, wv, ws, we):
    n = x.shape[0]
    return pl.pallas_call(
        _prep_body,
        out_shape=[
            jax.ShapeDtypeStruct((n, 128), jnp.float32),
            jax.ShapeDtypeStruct((n, 128), jnp.float32),
            jax.ShapeDtypeStruct((n, _D), jnp.float32),
        ],
    )(x, wq, wk, wv, ws, we)


def _post_mid_body(acc_ref, s_ref, we_ref, jk_ref,
                   wq_ref, wk_ref, wv_ref, ws_ref, we2_ref,
                   jko_ref, kv_ref, qq_ref, s2_ref):
    acc = acc_ref[0, :_N] + acc_ref[1, :_N]
    aggv = acc[:, :_D]
    agge = acc[:, _D:_D + _EDIM]
    den = acc[:, _D + _EDIM:_D + _EDIM + 1]
    agg = (aggv + agge @ we_ref[...]) / (den + 1e-16)
    pre = agg + s_ref[...]
    out = jnp.where(pre > 0, pre, (jnp.exp(pre) - 1.0))
    jko_ref[...] = jnp.maximum(jk_ref[...], out)
    q = out @ wq_ref[...]
    kv_ref[:, :_D] = out @ wk_ref[...]
    kv_ref[:, _D:] = out @ wv_ref[...]
    qq_ref[:, :_D] = q
    qq_ref[:, _D:_D + _EDIM] = lax.dot_general(
        q, we2_ref[...], (((1,), (1,)), ((), ())))
    qq_ref[:, _D + _EDIM:] = jnp.zeros((out.shape[0], 128 - _D - _EDIM),
                                       jnp.float32)
    s2_ref[...] = out @ ws_ref[...]


def _tc_post_mid(acc, s, we, jk, wq, wk, wv, ws, we2):
    return pl.pallas_call(
        _post_mid_body,
        out_shape=[
            jax.ShapeDtypeStruct((_N, _D), jnp.float32),
            jax.ShapeDtypeStruct((_N, 128), jnp.float32),
            jax.ShapeDtypeStruct((_N, 128), jnp.float32),
            jax.ShapeDtypeStruct((_N, _D), jnp.float32),
        ],
    )(acc, s, we, jk, wq, wk, wv, ws, we2)


def _post_last_body(acc_ref, s_ref, we_ref, jk_ref, jko_ref):
    acc = acc_ref[0, :_N] + acc_ref[1, :_N]
    aggv = acc[:, :_D]
    agge = acc[:, _D:_D + _EDIM]
    den = acc[:, _D + _EDIM:_D + _EDIM + 1]
    agg = (aggv + agge @ we_ref[...]) / (den + 1e-16)
    pre = agg + s_ref[...]
    out = jnp.where(pre > 0, pre, (jnp.exp(pre) - 1.0))
    jko_ref[...] = jnp.maximum(jk_ref[...], out)


def _tc_post_last(acc, s, we, jk):
    return pl.pallas_call(
        _post_last_body,
        out_shape=jax.ShapeDtypeStruct((_N, _D), jnp.float32),
    )(acc, s, we, jk)


def _pool_body(jk_ref, bt_ref,
               gpw1_ref, gpb1_ref, gpw2_ref, gpb2_ref,
               gbw1_ref, gbb1_ref, gbw2_ref, gbb2_ref,
               mw0_ref, mb0_ref, mw1_ref, mb1_ref,
               mw2_ref, mb2_ref, mw3_ref, mb3_ref,
               o_ref):
    jk = jk_ref[...]
    bt = bt_ref[...]
    gids = lax.broadcasted_iota(jnp.int32, (_G, _N), 0)
    oht = (gids == bt).astype(jnp.float32)

    def pool(w1, b1, w2, b2):
        hid = jnp.maximum(jk @ w1 + b1, 0.0)
        gate = hid @ w2 + b2
        ex = jnp.exp(gate)
        num = oht @ (jk * ex)
        den = oht @ ex
        return num / (den + 1e-16)

    gp = pool(gpw1_ref[...], gpb1_ref[...], gpw2_ref[...], gpb2_ref[...])
    gb = pool(gbw1_ref[...], gbb1_ref[...], gbw2_ref[...], gbb2_ref[...])
    h = jnp.concatenate([gp, gb], axis=1)
    h = jnp.maximum(h @ mw0_ref[...] + mb0_ref[...], 0.0)
    h = jnp.maximum(h @ mw1_ref[...] + mb1_ref[...], 0.0)
    h = jnp.maximum(h @ mw2_ref[...] + mb2_ref[...], 0.0)
    o_ref[...] = h @ mw3_ref[...] + mb3_ref[...]


def _tc_pool(jk, bt, gpw1, gpb1, gpw2, gpb2, gbw1, gbb1, gbw2, gbb2,
             mw0, mb0, mw1, mb1, mw2, mb2, mw3, mb3):
    return pl.pallas_call(
        _pool_body,
        out_shape=jax.ShapeDtypeStruct((_G, 1), jnp.float32),
    )(jk, bt, gpw1, gpb1, gpw2, gpb2, gbw1, gbb1, gbw2, gbb2,
      mw0, mb0, mw1, mb1, mw2, mb2, mw3, mb3)


def _edge_body(kv_hbm, qq_hbm, src_hbm, dst_hbm, ea_hbm, acc_hbm,
               src_v, dst_v, ea_v, kv_b, qq_b, out_b, zb, acc_sh,
               sem1, sem2):
    c = lax.axis_index("c")
    s = lax.axis_index("s")
    wid = s * 2 + c
    ebase = wid * _TPW

    iota16 = lax.broadcasted_iota(jnp.int32, (16,), 0)
    dmask = jnp.where(iota16 == 0, 1.0, 0.0).astype(jnp.float32)
    zeros16 = jnp.zeros((16,), jnp.float32)
    gdn = lax.GatherDimensionNumbers(
        offset_dims=(), collapsed_slice_dims=(0,), start_index_map=(0,))

    def _allsum(a):
        for k in (8, 4, 2, 1):
            a = a + lax.gather(a, (iota16 ^ k)[:, None], gdn, (1,),
                               mode=lax.GatherScatterMode.PROMISE_IN_BOUNDS)
        return a

    def _z1(i, _):
        def _z2(j, _):
            zb[i, pl.ds(j * 16, 16)] = zeros16
            return 0
        lax.fori_loop(0, 8, _z2, 0)
        return 0
    lax.fori_loop(0, _ZR, _z1, 0)

    rbase = s * _RPT
    for t in range(_RPT // _ZR):
        pltpu.sync_copy(zb, acc_sh.at[pl.ds(rbase + t * _ZR, _ZR)])

    def _z3(i, _):
        out_b[i, pl.ds(96, 16)] = zeros16
        out_b[i, pl.ds(112, 16)] = zeros16
        return 0
    lax.fori_loop(0, _NB, _z3, 0)

    plsc.subcore_barrier()

    def step(j, _):
        base = ebase + j * _NB
        pltpu.sync_copy(src_hbm.at[pl.ds(base, _NB)], src_v)
        pltpu.sync_copy(dst_hbm.at[pl.ds(base, _NB)], dst_v)
        pltpu.sync_copy(ea_hbm.at[pl.ds(base * _EDIM, _NB * _EDIM)], ea_v)
        cp1 = pltpu.async_copy(kv_hbm.at[src_v], kv_b, sem1)
        cp2 = pltpu.async_copy(qq_hbm.at[dst_v], qq_b, sem2)
        cp1.wait()
        cp2.wait()

        def edge(e, _):
            a = kv_b[e, pl.ds(0, 16)] * qq_b[e, pl.ds(0, 16)]
            a = a + kv_b[e, pl.ds(16, 16)] * qq_b[e, pl.ds(16, 16)]
            a = a + kv_b[e, pl.ds(32, 16)] * qq_b[e, pl.ds(32, 16)]
            a = a + kv_b[e, pl.ds(48, 16)] * qq_b[e, pl.ds(48, 16)]
            eav = ea_v[pl.ds(e * _EDIM, 16)]
            a = a + qq_b[e, pl.ds(64, 16)] * eav
            exv = jnp.exp(_allsum(a) * 0.125)
            out_b[e, pl.ds(0, 16)] = exv * kv_b[e, pl.ds(64, 16)]
            out_b[e, pl.ds(16, 16)] = exv * kv_b[e, pl.ds(80, 16)]
            out_b[e, pl.ds(32, 16)] = exv * kv_b[e, pl.ds(96, 16)]
            out_b[e, pl.ds(48, 16)] = exv * kv_b[e, pl.ds(112, 16)]
            out_b[e, pl.ds(64, 16)] = exv * eav
            out_b[e, pl.ds(80, 16)] = exv * dmask
            return 0

        lax.fori_loop(0, _NB, edge, 0)
        pltpu.sync_copy(out_b, acc_sh.at[dst_v], add=True)
        return 0

    lax.fori_loop(0, _NSTEP, step, 0)

    plsc.subcore_barrier()

    for t in range(_RPT // _ZR):
        pltpu.sync_copy(acc_sh.at[pl.ds(rbase + t * _ZR, _ZR)], zb)
        pltpu.sync_copy(zb, acc_hbm.at[c, pl.ds(rbase + t * _ZR, _ZR)])


def _sc_edge(kv, qq, src, dst, ea_flat):
    mesh = plsc.VectorSubcoreMesh(core_axis_name="c", subcore_axis_name="s")
    f = pl.kernel(
        _edge_body,
        out_type=jax.ShapeDtypeStruct((2, _NPAD, 128), jnp.float32),
        mesh=mesh,
        scratch_types=[
            pltpu.VMEM((_NB,), jnp.int32),
            pltpu.VMEM((_NB,), jnp.int32),
            pltpu.VMEM((_NB * _EDIM,), jnp.float32),
            pltpu.VMEM((_NB, 128), jnp.float32),
            pltpu.VMEM((_NB, 128), jnp.float32),
            pltpu.VMEM((_NB, 128), jnp.float32),
            pltpu.VMEM((_ZR, 128), jnp.float32),
            pltpu.VMEM_SHARED((_NPAD, 128), jnp.float32),
            pltpu.SemaphoreType.DMA,
            pltpu.SemaphoreType.DMA,
        ],
    )
    return f(kv, qq, src, dst, ea_flat)


def kernel(x, edge_index, edge_attr, batch, Wq0, Wk0, Wv0, We0, Ws0, cWq, cWk,
           cWv, cWe, cWs, GP_W1, GP_b1, GP_W2, GP_b2, GB_W1, GB_b1, GB_W2,
           GB_b2, M_W0, M_b0, M_W1, M_b1, M_W2, M_b2, M_W3, M_b3):
    src = edge_index[0]
    dst = edge_index[1]
    ea_flat = edge_attr.reshape(-1)
    bt = batch.reshape(1, _N)

    kv, qq, s = _tc_prep(x, Wq0, Wk0, Wv0, Ws0, We0)
    acc = _sc_edge(kv, qq, src, dst, ea_flat)
    jk0 = jnp.full((_N, _D), -jnp.inf, jnp.float32)
    jk, kv, qq, s = _tc_post_mid(acc, s, We0, jk0, cWq[0], cWk[0], cWv[0],
                                 cWs[0], cWe[0])
    for i in range(4):
        acc = _sc_edge(kv, qq, src, dst, ea_flat)
        jk, kv, qq, s = _tc_post_mid(acc, s, cWe[i], jk, cWq[i + 1],
                                     cWk[i + 1], cWv[i + 1], cWs[i + 1],
                                     cWe[i + 1])
    acc = _sc_edge(kv, qq, src, dst, ea_flat)
    jk = _tc_post_last(acc, s, cWe[4], jk)

    return _tc_pool(jk, bt,
                    GP_W1, GP_b1.reshape(1, _D), GP_W2, GP_b2.reshape(1, 1),
                    GB_W1, GB_b1.reshape(1, _D), GB_W2, GB_b2.reshape(1, 1),
                    M_W0, M_b0.reshape(1, 32), M_W1, M_b1.reshape(1, 16),
                    M_W2, M_b2.reshape(1, 8), M_W3, M_b3.reshape(1, 1))


# speedup vs baseline: 16.5857x; 1.6395x over previous
import functools

import jax
import jax.numpy as jnp
from jax import lax
from jax.experimental import pallas as pl
from jax.experimental.pallas import tpu as pltpu
from jax.experimental.pallas import tpu_sc as plsc

_N = 10000
_E = 320000
_D = 64
_EDIM = 16
_G = 16
_NTILES = 32
_TPW = _E // _NTILES
_NB = 80
_NSTEP = _TPW // _NB
_NPAD = 10240
_RPT = _NPAD // 16
_ZR = 128


def _prep_body(x_ref, wq_ref, wk_ref, wv_ref, ws_ref, we_ref,
               kv_ref, qq_ref, s_ref):
    x = x_ref[...]
    q = x @ wq_ref[...]
    kv_ref[:, :_D] = x @ wk_ref[...]
    kv_ref[:, _D:] = x @ wv_ref[...]
    qq_ref[:, :_D] = q
    qq_ref[:, _D:_D + _EDIM] = lax.dot_general(
        q, we_ref[...], (((1,), (1,)), ((), ())))
    qq_ref[:, _D + _EDIM:] = jnp.zeros((x.shape[0], 128 - _D - _EDIM),
                                       jnp.float32)
    s_ref[...] = x @ ws_ref[...]


def _tc_prep(x, wq, wk, wv, ws, we):
    n = x.shape[0]
    return pl.pallas_call(
        _prep_body,
        out_shape=[
            jax.ShapeDtypeStruct((n, 128), jnp.float32),
            jax.ShapeDtypeStruct((n, 128), jnp.float32),
            jax.ShapeDtypeStruct((n, _D), jnp.float32),
        ],
    )(x, wq, wk, wv, ws, we)


def _post_mid_body(acc_ref, s_ref, we_ref, jk_ref,
                   wq_ref, wk_ref, wv_ref, ws_ref, we2_ref,
                   jko_ref, kv_ref, qq_ref, s2_ref):
    acc = acc_ref[0, :_N] + acc_ref[1, :_N]
    aggv = acc[:, :_D]
    agge = acc[:, _D:_D + _EDIM]
    den = acc[:, _D + _EDIM:_D + _EDIM + 1]
    agg = (aggv + agge @ we_ref[...]) / (den + 1e-16)
    pre = agg + s_ref[...]
    out = jnp.where(pre > 0, pre, (jnp.exp(pre) - 1.0))
    jko_ref[...] = jnp.maximum(jk_ref[...], out)
    q = out @ wq_ref[...]
    kv_ref[:, :_D] = out @ wk_ref[...]
    kv_ref[:, _D:] = out @ wv_ref[...]
    qq_ref[:, :_D] = q
    qq_ref[:, _D:_D + _EDIM] = lax.dot_general(
        q, we2_ref[...], (((1,), (1,)), ((), ())))
    qq_ref[:, _D + _EDIM:] = jnp.zeros((out.shape[0], 128 - _D - _EDIM),
                                       jnp.float32)
    s2_ref[...] = out @ ws_ref[...]


def _tc_post_mid(acc, s, we, jk, wq, wk, wv, ws, we2):
    return pl.pallas_call(
        _post_mid_body,
        out_shape=[
            jax.ShapeDtypeStruct((_N, _D), jnp.float32),
            jax.ShapeDtypeStruct((_N, 128), jnp.float32),
            jax.ShapeDtypeStruct((_N, 128), jnp.float32),
            jax.ShapeDtypeStruct((_N, _D), jnp.float32),
        ],
    )(acc, s, we, jk, wq, wk, wv, ws, we2)


def _post_last_body(acc_ref, s_ref, we_ref, jk_ref, jko_ref):
    acc = acc_ref[0, :_N] + acc_ref[1, :_N]
    aggv = acc[:, :_D]
    agge = acc[:, _D:_D + _EDIM]
    den = acc[:, _D + _EDIM:_D + _EDIM + 1]
    agg = (aggv + agge @ we_ref[...]) / (den + 1e-16)
    pre = agg + s_ref[...]
    out = jnp.where(pre > 0, pre, (jnp.exp(pre) - 1.0))
    jko_ref[...] = jnp.maximum(jk_ref[...], out)


def _tc_post_last(acc, s, we, jk):
    return pl.pallas_call(
        _post_last_body,
        out_shape=jax.ShapeDtypeStruct((_N, _D), jnp.float32),
    )(acc, s, we, jk)


def _pool_body(jk_ref, bt_ref,
               gpw1_ref, gpb1_ref, gpw2_ref, gpb2_ref,
               gbw1_ref, gbb1_ref, gbw2_ref, gbb2_ref,
               mw0_ref, mb0_ref, mw1_ref, mb1_ref,
               mw2_ref, mb2_ref, mw3_ref, mb3_ref,
               o_ref):
    jk = jk_ref[...]
    bt = bt_ref[...]
    gids = lax.broadcasted_iota(jnp.int32, (_G, _N), 0)
    oht = (gids == bt).astype(jnp.float32)

    def pool(w1, b1, w2, b2):
        hid = jnp.maximum(jk @ w1 + b1, 0.0)
        gate = hid @ w2 + b2
        ex = jnp.exp(gate)
        num = oht @ (jk * ex)
        den = oht @ ex
        return num / (den + 1e-16)

    gp = pool(gpw1_ref[...], gpb1_ref[...], gpw2_ref[...], gpb2_ref[...])
    gb = pool(gbw1_ref[...], gbb1_ref[...], gbw2_ref[...], gbb2_ref[...])
    h = jnp.concatenate([gp, gb], axis=1)
    h = jnp.maximum(h @ mw0_ref[...] + mb0_ref[...], 0.0)
    h = jnp.maximum(h @ mw1_ref[...] + mb1_ref[...], 0.0)
    h = jnp.maximum(h @ mw2_ref[...] + mb2_ref[...], 0.0)
    o_ref[...] = h @ mw3_ref[...] + mb3_ref[...]


def _tc_pool(jk, bt, gpw1, gpb1, gpw2, gpb2, gbw1, gbb1, gbw2, gbb2,
             mw0, mb0, mw1, mb1, mw2, mb2, mw3, mb3):
    return pl.pallas_call(
        _pool_body,
        out_shape=jax.ShapeDtypeStruct((_G, 1), jnp.float32),
    )(jk, bt, gpw1, gpb1, gpw2, gpb2, gbw1, gbb1, gbw2, gbb2,
      mw0, mb0, mw1, mb1, mw2, mb2, mw3, mb3)


def _edge_body(kv_hbm, qq_hbm, src_hbm, dst_hbm, ea_hbm, acc_hbm,
               src_v, dst_v, ea_v, kv_b, qq_b, out_b, zb, acc_sh,
               sem1, sem2):
    c = lax.axis_index("c")
    s = lax.axis_index("s")
    wid = s * 2 + c
    ebase = wid * _TPW

    iota16 = lax.broadcasted_iota(jnp.int32, (16,), 0)
    dmask = jnp.where(iota16 == 0, 1.0, 0.0).astype(jnp.float32)
    zeros16 = jnp.zeros((16,), jnp.float32)
    gdn = lax.GatherDimensionNumbers(
        offset_dims=(), collapsed_slice_dims=(0,), start_index_map=(0,))

    def _allsum(a):
        for k in (8, 4, 2, 1):
            a = a + lax.gather(a, (iota16 ^ k)[:, None], gdn, (1,),
                               mode=lax.GatherScatterMode.PROMISE_IN_BOUNDS)
        return a

    def _z1(i, _):
        def _z2(j, _):
            zb[i, pl.ds(j * 16, 16)] = zeros16
            return 0
        lax.fori_loop(0, 8, _z2, 0)
        return 0
    lax.fori_loop(0, _ZR, _z1, 0)

    rbase = s * _RPT
    for t in range(_RPT // _ZR):
        pltpu.sync_copy(zb, acc_sh.at[pl.ds(rbase + t * _ZR, _ZR)])

    def _z3(i, _):
        out_b[i, pl.ds(96, 16)] = zeros16
        out_b[i, pl.ds(112, 16)] = zeros16
        return 0
    lax.fori_loop(0, _NB, _z3, 0)

    plsc.subcore_barrier()

    def step(j, _):
        base = ebase + j * _NB
        ci = pltpu.async_copy(src_hbm.at[pl.ds(base, _NB)], src_v, sem1)
        cj = pltpu.async_copy(dst_hbm.at[pl.ds(base, _NB)], dst_v, sem1)
        ck = pltpu.async_copy(ea_hbm.at[pl.ds(base * _EDIM, _NB * _EDIM)],
                              ea_v, sem1)
        ci.wait()
        cj.wait()
        ck.wait()
        cp1 = pltpu.async_copy(kv_hbm.at[src_v], kv_b, sem1)
        cp2 = pltpu.async_copy(qq_hbm.at[dst_v], qq_b, sem2)
        cp1.wait()
        cp2.wait()

        @plsc.parallel_loop(0, _NB, 1, unroll=4)
        def edge(e):
            a = kv_b[e, pl.ds(0, 16)] * qq_b[e, pl.ds(0, 16)]
            a = a + kv_b[e, pl.ds(16, 16)] * qq_b[e, pl.ds(16, 16)]
            a = a + kv_b[e, pl.ds(32, 16)] * qq_b[e, pl.ds(32, 16)]
            a = a + kv_b[e, pl.ds(48, 16)] * qq_b[e, pl.ds(48, 16)]
            eav = ea_v[pl.ds(e * _EDIM, 16)]
            a = a + qq_b[e, pl.ds(64, 16)] * eav
            exv = jnp.exp(_allsum(a) * 0.125)
            out_b[e, pl.ds(0, 16)] = exv * kv_b[e, pl.ds(64, 16)]
            out_b[e, pl.ds(16, 16)] = exv * kv_b[e, pl.ds(80, 16)]
            out_b[e, pl.ds(32, 16)] = exv * kv_b[e, pl.ds(96, 16)]
            out_b[e, pl.ds(48, 16)] = exv * kv_b[e, pl.ds(112, 16)]
            out_b[e, pl.ds(64, 16)] = exv * eav
            out_b[e, pl.ds(80, 16)] = exv * dmask

        pltpu.sync_copy(out_b, acc_sh.at[dst_v], add=True)
        return 0

    lax.fori_loop(0, _NSTEP, step, 0)

    plsc.subcore_barrier()

    for t in range(_RPT // _ZR):
        pltpu.sync_copy(acc_sh.at[pl.ds(rbase + t * _ZR, _ZR)], zb)
        pltpu.sync_copy(zb, acc_hbm.at[c, pl.ds(rbase + t * _ZR, _ZR)])


def _sc_edge(kv, qq, src, dst, ea_flat):
    mesh = plsc.VectorSubcoreMesh(core_axis_name="c", subcore_axis_name="s")
    f = pl.kernel(
        _edge_body,
        out_type=jax.ShapeDtypeStruct((2, _NPAD, 128), jnp.float32),
        mesh=mesh,
        scratch_types=[
            pltpu.VMEM((_NB,), jnp.int32),
            pltpu.VMEM((_NB,), jnp.int32),
            pltpu.VMEM((_NB * _EDIM,), jnp.float32),
            pltpu.VMEM((_NB, 128), jnp.float32),
            pltpu.VMEM((_NB, 128), jnp.float32),
            pltpu.VMEM((_NB, 128), jnp.float32),
            pltpu.VMEM((_ZR, 128), jnp.float32),
            pltpu.VMEM_SHARED((_NPAD, 128), jnp.float32),
            pltpu.SemaphoreType.DMA,
            pltpu.SemaphoreType.DMA,
        ],
    )
    return f(kv, qq, src, dst, ea_flat)


def kernel(x, edge_index, edge_attr, batch, Wq0, Wk0, Wv0, We0, Ws0, cWq, cWk,
           cWv, cWe, cWs, GP_W1, GP_b1, GP_W2, GP_b2, GB_W1, GB_b1, GB_W2,
           GB_b2, M_W0, M_b0, M_W1, M_b1, M_W2, M_b2, M_W3, M_b3):
    src = edge_index[0]
    dst = edge_index[1]
    ea_flat = edge_attr.reshape(-1)
    bt = batch.reshape(1, _N)

    kv, qq, s = _tc_prep(x, Wq0, Wk0, Wv0, Ws0, We0)
    acc = _sc_edge(kv, qq, src, dst, ea_flat)
    jk0 = jnp.full((_N, _D), -jnp.inf, jnp.float32)
    jk, kv, qq, s = _tc_post_mid(acc, s, We0, jk0, cWq[0], cWk[0], cWv[0],
                                 cWs[0], cWe[0])
    for i in range(4):
        acc = _sc_edge(kv, qq, src, dst, ea_flat)
        jk, kv, qq, s = _tc_post_mid(acc, s, cWe[i], jk, cWq[i + 1],
                                     cWk[i + 1], cWv[i + 1], cWs[i + 1],
                                     cWe[i + 1])
    acc = _sc_edge(kv, qq, src, dst, ea_flat)
    jk = _tc_post_last(acc, s, cWe[4], jk)

    return _tc_pool(jk, bt,
                    GP_W1, GP_b1.reshape(1, _D), GP_W2, GP_b2.reshape(1, 1),
                    GB_W1, GB_b1.reshape(1, _D), GB_W2, GB_b2.reshape(1, 1),
                    M_W0, M_b0.reshape(1, 32), M_W1, M_b1.reshape(1, 16),
                    M_W2, M_b2.reshape(1, 8), M_W3, M_b3.reshape(1, 1))
